# Initial kernel scaffold; baseline (speedup 1.0000x reference)
import jax
import jax.numpy as jnp
from jax.experimental import pallas as pl


def kernel(x, edge_index, candidates, action, machine_state, g0w1, g0b1, g0w2, g0b2, g1w1, g1b1, g1w2, g1b2, g2w1, g2b1, g2w2, g2b2, aw1, ab1, aw2, ab2, aw3, ab3, cw1, cb1, cw2, cb2):
    raise NotImplementedError("write your pallas kernel here")


# speedup vs baseline: 71.7094x; 71.7094x over previous
import functools

import jax
import jax.numpy as jnp
from jax import lax
from jax.experimental import pallas as pl
from jax.experimental.pallas import tpu as pltpu
from jax.experimental.pallas import tpu_sc as plsc

NB = 4
NN = 10000
NE = 320000
DD = 128
CH = 32

TILES = 16
EPT = NE // TILES
C = 100
NCH = EPT // C
SLOTS = 2
SEG = 40
NSEG = NCH // SEG
DRC = 80
NDC = NN // DRC
NQD = -(-NDC // TILES)

BN = 1000
NBJ = NN // BN

@functools.cache
def _build_segsum_sc():
    mesh = plsc.VectorSubcoreMesh(core_axis_name="c", subcore_axis_name="s")

    @functools.partial(
        pl.kernel,
        mesh=mesh,
        out_type=jax.ShapeDtypeStruct((NB, NN, DD), jnp.float32),
        scratch_types=[
            pltpu.VMEM((SEG, C), jnp.int32),
            pltpu.VMEM((SEG, C), jnp.int32),
            pltpu.VMEM((C, DD), jnp.float32),
            pltpu.VMEM((C, DD), jnp.float32),
            pltpu.VMEM((DRC, DD), jnp.float32),
            pltpu.VMEM_SHARED((NN, DD), jnp.float32),
            pltpu.SemaphoreType.DMA,
            pltpu.SemaphoreType.DMA,
        ],
    )
    def segsum_sc(h_hbm, srcg_hbm, dst_hbm, z_hbm, out_hbm,
                  sidx, didx, s0, s1, stg, acc, m0, m1):
        c = lax.axis_index("c")
        t = lax.axis_index("s")
        slots = (s0, s1)
        sems = (m0, m1)

        for r in range(2):
            b = c + 2 * r

            pltpu.sync_copy(z_hbm, stg)
            for q in range(NQD):
                cid = t + TILES * q

                @pl.when(cid < NDC)
                def _():
                    pltpu.sync_copy(stg, acc.at[pl.ds(cid * DRC, DRC)])
            plsc.subcore_barrier()

            for seg in range(NSEG):
                pltpu.sync_copy(srcg_hbm.at[b, t, seg], sidx)
                pltpu.sync_copy(dst_hbm.at[t, seg], didx)

                for s in range(SLOTS):
                    pltpu.async_copy(h_hbm.at[sidx.at[s]], slots[s], sems[s])

                def group(o, carry):
                    for s in range(SLOTS):
                        j = o * SLOTS + s
                        pltpu.make_async_copy(
                            h_hbm.at[sidx.at[j]], slots[s], sems[s]).wait()
                        pltpu.sync_copy(slots[s], acc.at[didx.at[j]], add=True)

                        @pl.when(o < SEG // SLOTS - 1)
                        def _():
                            pltpu.async_copy(
                                h_hbm.at[sidx.at[j + SLOTS]], slots[s], sems[s])
                    return carry

                lax.fori_loop(0, SEG // SLOTS, group, 0)
            plsc.subcore_barrier()

            for q in range(NQD):
                cid = t + TILES * q

                @pl.when(cid < NDC)
                def _():
                    pltpu.sync_copy(acc.at[pl.ds(cid * DRC, DRC)], stg)
                    pltpu.sync_copy(stg, out_hbm.at[b, pl.ds(cid * DRC, DRC)])
            plsc.subcore_barrier()

    return segsum_sc


def _segsum(h2d, srcg, dstt, zrow):
    return _build_segsum_sc()(h2d, srcg, dstt, zrow)


def _gin_body(h_ref, a_ref, w1_ref, b1_ref, w2_ref, b2_ref, o_ref, g_ref):
    j = pl.program_id(1)
    z = h_ref[0] + a_ref[0]
    z = jnp.maximum(
        jnp.dot(z, w1_ref[...], preferred_element_type=jnp.float32) + b1_ref[...], 0.0)
    o = jnp.dot(z, w2_ref[...], preferred_element_type=jnp.float32) + b2_ref[...]
    o_ref[0] = o

    @pl.when(j == 0)
    def _():
        g_ref[...] = jnp.zeros_like(g_ref)

    g_ref[0] += jnp.sum(o, axis=0, keepdims=True) * (1.0 / NN)


_gin = pl.pallas_call(
    _gin_body,
    grid=(NB, NBJ),
    in_specs=[
        pl.BlockSpec((1, BN, DD), lambda b, j: (b, j, 0)),
        pl.BlockSpec((1, BN, DD), lambda b, j: (b, j, 0)),
        pl.BlockSpec((DD, DD), lambda b, j: (0, 0)),
        pl.BlockSpec((1, DD), lambda b, j: (0, 0)),
        pl.BlockSpec((DD, DD), lambda b, j: (0, 0)),
        pl.BlockSpec((1, DD), lambda b, j: (0, 0)),
    ],
    out_specs=[
        pl.BlockSpec((1, BN, DD), lambda b, j: (b, j, 0)),
        pl.BlockSpec((1, 1, DD), lambda b, j: (b, 0, 0)),
    ],
    out_shape=[
        jax.ShapeDtypeStruct((NB, NN, DD), jnp.float32),
        jax.ShapeDtypeStruct((NB, 1, DD), jnp.float32),
    ],
)


def _scores_body(h_ref, g_ref, ms_ref, aw1_ref, ab1_ref, aw2_ref, ab2_ref,
                 aw3_ref, ab3_ref, o_ref):
    be = (ab1_ref[...]
          + jnp.dot(g_ref[0], aw1_ref[DD:2 * DD, :],
                    preferred_element_type=jnp.float32)
          + jnp.dot(ms_ref[...], aw1_ref[2 * DD:3 * DD, :],
                    preferred_element_type=jnp.float32))
    t1 = jnp.maximum(
        jnp.dot(h_ref[0], aw1_ref[0:DD, :], preferred_element_type=jnp.float32) + be,
        0.0)
    t2 = jnp.maximum(
        jnp.dot(t1, aw2_ref[...], preferred_element_type=jnp.float32) + ab2_ref[...],
        0.0)
    s = (jnp.sum(t2 * aw3_ref[...], axis=1) + ab3_ref[0]) * 10.0
    o_ref[0, 0] = s


_scores = pl.pallas_call(
    _scores_body,
    grid=(NB, NBJ),
    in_specs=[
        pl.BlockSpec((1, BN, DD), lambda b, j: (b, j, 0)),
        pl.BlockSpec((1, 1, DD), lambda b, j: (b, 0, 0)),
        pl.BlockSpec((1, DD), lambda b, j: (0, 0)),
        pl.BlockSpec((3 * DD, DD), lambda b, j: (0, 0)),
        pl.BlockSpec((1, DD), lambda b, j: (0, 0)),
        pl.BlockSpec((DD, DD), lambda b, j: (0, 0)),
        pl.BlockSpec((1, DD), lambda b, j: (0, 0)),
        pl.BlockSpec((1, DD), lambda b, j: (0, 0)),
        pl.BlockSpec(memory_space=pltpu.SMEM),
    ],
    out_specs=pl.BlockSpec((1, 1, BN), lambda b, j: (b * NBJ + j, 0, 0)),
    out_shape=jax.ShapeDtypeStruct((NB * NBJ, 1, BN), jnp.float32),
)


def _final_body(s_ref, cand_ref, act_ref, g_ref, cw1_ref, cb1_ref, cw2_ref,
                cb2_ref, lp_ref, ent_ref, v_ref):
    fmin = jnp.finfo(jnp.float32).min
    idxs = lax.broadcasted_iota(jnp.int32, (1, NN), 1)
    for b in range(NB):
        s = s_ref[b:b + 1, :]
        m = jnp.max(s, axis=1, keepdims=True)
        e = jnp.exp(s - m)
        p0 = e / jnp.sum(e, axis=1, keepdims=True)
        valid = cand_ref[b:b + 1, :] > 0
        logits = jnp.where(valid, p0, -jnp.inf)
        m2 = jnp.max(logits, axis=1, keepdims=True)
        e2 = jnp.exp(logits - m2)
        z2 = jnp.sum(e2, axis=1, keepdims=True)
        logp = logits - m2 - jnp.log(z2)
        pfull = e2 / z2
        ab = act_ref[b]
        lp = jnp.sum(jnp.where(idxs == ab, logp, 0.0), axis=1, keepdims=True)
        ent = -jnp.sum(pfull * jnp.maximum(logp, fmin), axis=1, keepdims=True)
        gb = g_ref[b:b + 1, :]
        tc = jnp.maximum(
            jnp.dot(gb, cw1_ref[...], preferred_element_type=jnp.float32)
            + cb1_ref[...], 0.0)
        v = jnp.sum(tc * cw2_ref[...], axis=1, keepdims=True) + cb2_ref[0]
        lp_ref[b:b + 1, :] = jnp.broadcast_to(lp, (1, DD))
        ent_ref[b:b + 1, :] = jnp.broadcast_to(ent, (1, DD))
        v_ref[b:b + 1, :] = jnp.broadcast_to(v, (1, DD))


_final = pl.pallas_call(
    _final_body,
    in_specs=[
        pl.BlockSpec(memory_space=pltpu.VMEM),
        pl.BlockSpec(memory_space=pltpu.VMEM),
        pl.BlockSpec(memory_space=pltpu.SMEM),
        pl.BlockSpec(memory_space=pltpu.VMEM),
        pl.BlockSpec(memory_space=pltpu.VMEM),
        pl.BlockSpec(memory_space=pltpu.VMEM),
        pl.BlockSpec(memory_space=pltpu.VMEM),
        pl.BlockSpec(memory_space=pltpu.SMEM),
    ],
    out_shape=[
        jax.ShapeDtypeStruct((NB, DD), jnp.float32),
        jax.ShapeDtypeStruct((NB, DD), jnp.float32),
        jax.ShapeDtypeStruct((NB, DD), jnp.float32),
    ],
)


def kernel(x, edge_index, candidates, action, machine_state,
           g0w1, g0b1, g0w2, g0b2, g1w1, g1b1, g1w2, g1b2,
           g2w1, g2b1, g2w2, g2b2, aw1, ab1, aw2, ab2, aw3, ab3,
           cw1, cb1, cw2, cb2):
    src = edge_index[0]
    dst = edge_index[1]
    offs = (jnp.arange(NB, dtype=jnp.int32) * NN)[:, None]
    srcg = (src[None, :] + offs).reshape(NB, TILES, NSEG, SEG, C)
    dstt = dst.reshape(TILES, NSEG, SEG, C)
    zrow = jnp.zeros((DRC, DD), jnp.float32)
    ms = machine_state.reshape(1, DD)

    h = x
    gmean = None
    for (w1, b1, w2, b2) in ((g0w1, g0b1, g0w2, g0b2),
                             (g1w1, g1b1, g1w2, g1b2),
                             (g2w1, g2b1, g2w2, g2b2)):
        agg = _segsum(h.reshape(NB * NN, DD), srcg, dstt, zrow)
        h, gmean = _gin(h, agg, w1, b1.reshape(1, DD), w2, b2.reshape(1, DD))

    s3 = _scores(h, gmean, ms, aw1, ab1.reshape(1, DD), aw2,
                 ab2.reshape(1, DD), aw3.reshape(1, DD), ab3)
    lps, ents, vs = _final(s3.reshape(NB, NN), candidates, action,
                           gmean.reshape(NB, DD), cw1, cb1.reshape(1, CH),
                           cw2.reshape(1, CH), cb2)
    return action, lps[:, 0], ents[:, 0], vs[:, 0]


# speedup vs baseline: 76.1211x; 1.0615x over previous
import functools

import jax
import jax.numpy as jnp
from jax import lax
from jax.experimental import pallas as pl
from jax.experimental.pallas import tpu as pltpu
from jax.experimental.pallas import tpu_sc as plsc

NB = 4
NN = 10000
NE = 320000
DD = 128
CH = 32

TILES = 16
EPT = NE // TILES
C = 50
NCH = EPT // C
SLOTS = 4
SEG = 40
NSEG = NCH // SEG
DRC = 80
NDC = NN // DRC
NQD = -(-NDC // TILES)

BN = 1000
NBJ = NN // BN

@functools.cache
def _build_segsum_sc():
    mesh = plsc.VectorSubcoreMesh(core_axis_name="c", subcore_axis_name="s")

    @functools.partial(
        pl.kernel,
        mesh=mesh,
        out_type=jax.ShapeDtypeStruct((NB, NN, DD), jnp.float32),
        scratch_types=[
            pltpu.VMEM((SEG, C), jnp.int32),
            pltpu.VMEM((SEG, C), jnp.int32),
            pltpu.VMEM((C, DD), jnp.float32),
            pltpu.VMEM((C, DD), jnp.float32),
            pltpu.VMEM((C, DD), jnp.float32),
            pltpu.VMEM((C, DD), jnp.float32),
            pltpu.VMEM((DRC, DD), jnp.float32),
            pltpu.VMEM_SHARED((NN, DD), jnp.float32),
            pltpu.SemaphoreType.DMA,
            pltpu.SemaphoreType.DMA,
            pltpu.SemaphoreType.DMA,
            pltpu.SemaphoreType.DMA,
        ],
    )
    def segsum_sc(h_hbm, srcg_hbm, dst_hbm, z_hbm, out_hbm,
                  sidx, didx, s0, s1, s2, s3, stg, acc, m0, m1, m2, m3):
        c = lax.axis_index("c")
        t = lax.axis_index("s")
        slots = (s0, s1, s2, s3)
        sems = (m0, m1, m2, m3)

        for r in range(2):
            b = c + 2 * r

            pltpu.sync_copy(z_hbm, stg)
            for q in range(NQD):
                cid = t + TILES * q

                @pl.when(cid < NDC)
                def _():
                    pltpu.sync_copy(stg, acc.at[pl.ds(cid * DRC, DRC)])
            plsc.subcore_barrier()

            for seg in range(NSEG):
                pltpu.sync_copy(srcg_hbm.at[b, t, seg], sidx)
                pltpu.sync_copy(dst_hbm.at[t, seg], didx)

                for s in range(SLOTS):
                    pltpu.async_copy(h_hbm.at[sidx.at[s]], slots[s], sems[s])

                def group(o, carry):
                    for s in range(SLOTS):
                        j = o * SLOTS + s
                        pltpu.make_async_copy(
                            h_hbm.at[sidx.at[j]], slots[s], sems[s]).wait()
                        pltpu.sync_copy(slots[s], acc.at[didx.at[j]], add=True)

                        @pl.when(o < SEG // SLOTS - 1)
                        def _():
                            pltpu.async_copy(
                                h_hbm.at[sidx.at[j + SLOTS]], slots[s], sems[s])
                    return carry

                lax.fori_loop(0, SEG // SLOTS, group, 0)
            plsc.subcore_barrier()

            for q in range(NQD):
                cid = t + TILES * q

                @pl.when(cid < NDC)
                def _():
                    pltpu.sync_copy(acc.at[pl.ds(cid * DRC, DRC)], stg)
                    pltpu.sync_copy(stg, out_hbm.at[b, pl.ds(cid * DRC, DRC)])
            plsc.subcore_barrier()

    return segsum_sc


def _segsum(h2d, srcg, dstt, zrow):
    return _build_segsum_sc()(h2d, srcg, dstt, zrow)


def _gin_body(h_ref, a_ref, w1_ref, b1_ref, w2_ref, b2_ref, o_ref, g_ref):
    j = pl.program_id(1)
    z = h_ref[0] + a_ref[0]
    z = jnp.maximum(
        jnp.dot(z, w1_ref[...], preferred_element_type=jnp.float32) + b1_ref[...], 0.0)
    o = jnp.dot(z, w2_ref[...], preferred_element_type=jnp.float32) + b2_ref[...]
    o_ref[0] = o

    @pl.when(j == 0)
    def _():
        g_ref[...] = jnp.zeros_like(g_ref)

    g_ref[0] += jnp.sum(o, axis=0, keepdims=True) * (1.0 / NN)


_gin = pl.pallas_call(
    _gin_body,
    grid=(NB, NBJ),
    in_specs=[
        pl.BlockSpec((1, BN, DD), lambda b, j: (b, j, 0)),
        pl.BlockSpec((1, BN, DD), lambda b, j: (b, j, 0)),
        pl.BlockSpec((DD, DD), lambda b, j: (0, 0)),
        pl.BlockSpec((1, DD), lambda b, j: (0, 0)),
        pl.BlockSpec((DD, DD), lambda b, j: (0, 0)),
        pl.BlockSpec((1, DD), lambda b, j: (0, 0)),
    ],
    out_specs=[
        pl.BlockSpec((1, BN, DD), lambda b, j: (b, j, 0)),
        pl.BlockSpec((1, 1, DD), lambda b, j: (b, 0, 0)),
    ],
    out_shape=[
        jax.ShapeDtypeStruct((NB, NN, DD), jnp.float32),
        jax.ShapeDtypeStruct((NB, 1, DD), jnp.float32),
    ],
)


def _scores_body(h_ref, g_ref, ms_ref, aw1_ref, ab1_ref, aw2_ref, ab2_ref,
                 aw3_ref, ab3_ref, o_ref):
    be = (ab1_ref[...]
          + jnp.dot(g_ref[0], aw1_ref[DD:2 * DD, :],
                    preferred_element_type=jnp.float32)
          + jnp.dot(ms_ref[...], aw1_ref[2 * DD:3 * DD, :],
                    preferred_element_type=jnp.float32))
    t1 = jnp.maximum(
        jnp.dot(h_ref[0], aw1_ref[0:DD, :], preferred_element_type=jnp.float32) + be,
        0.0)
    t2 = jnp.maximum(
        jnp.dot(t1, aw2_ref[...], preferred_element_type=jnp.float32) + ab2_ref[...],
        0.0)
    s = (jnp.sum(t2 * aw3_ref[...], axis=1) + ab3_ref[0]) * 10.0
    o_ref[0, 0] = s


_scores = pl.pallas_call(
    _scores_body,
    grid=(NB, NBJ),
    in_specs=[
        pl.BlockSpec((1, BN, DD), lambda b, j: (b, j, 0)),
        pl.BlockSpec((1, 1, DD), lambda b, j: (b, 0, 0)),
        pl.BlockSpec((1, DD), lambda b, j: (0, 0)),
        pl.BlockSpec((3 * DD, DD), lambda b, j: (0, 0)),
        pl.BlockSpec((1, DD), lambda b, j: (0, 0)),
        pl.BlockSpec((DD, DD), lambda b, j: (0, 0)),
        pl.BlockSpec((1, DD), lambda b, j: (0, 0)),
        pl.BlockSpec((1, DD), lambda b, j: (0, 0)),
        pl.BlockSpec(memory_space=pltpu.SMEM),
    ],
    out_specs=pl.BlockSpec((1, 1, BN), lambda b, j: (b * NBJ + j, 0, 0)),
    out_shape=jax.ShapeDtypeStruct((NB * NBJ, 1, BN), jnp.float32),
)


def _final_body(s_ref, cand_ref, act_ref, g_ref, cw1_ref, cb1_ref, cw2_ref,
                cb2_ref, lp_ref, ent_ref, v_ref):
    fmin = jnp.finfo(jnp.float32).min
    idxs = lax.broadcasted_iota(jnp.int32, (1, NN), 1)
    for b in range(NB):
        s = s_ref[b:b + 1, :]
        m = jnp.max(s, axis=1, keepdims=True)
        e = jnp.exp(s - m)
        p0 = e / jnp.sum(e, axis=1, keepdims=True)
        valid = cand_ref[b:b + 1, :] > 0
        logits = jnp.where(valid, p0, -jnp.inf)
        m2 = jnp.max(logits, axis=1, keepdims=True)
        e2 = jnp.exp(logits - m2)
        z2 = jnp.sum(e2, axis=1, keepdims=True)
        logp = logits - m2 - jnp.log(z2)
        pfull = e2 / z2
        ab = act_ref[b]
        lp = jnp.sum(jnp.where(idxs == ab, logp, 0.0), axis=1, keepdims=True)
        ent = -jnp.sum(pfull * jnp.maximum(logp, fmin), axis=1, keepdims=True)
        gb = g_ref[b:b + 1, :]
        tc = jnp.maximum(
            jnp.dot(gb, cw1_ref[...], preferred_element_type=jnp.float32)
            + cb1_ref[...], 0.0)
        v = jnp.sum(tc * cw2_ref[...], axis=1, keepdims=True) + cb2_ref[0]
        lp_ref[b:b + 1, :] = jnp.broadcast_to(lp, (1, DD))
        ent_ref[b:b + 1, :] = jnp.broadcast_to(ent, (1, DD))
        v_ref[b:b + 1, :] = jnp.broadcast_to(v, (1, DD))


_final = pl.pallas_call(
    _final_body,
    in_specs=[
        pl.BlockSpec(memory_space=pltpu.VMEM),
        pl.BlockSpec(memory_space=pltpu.VMEM),
        pl.BlockSpec(memory_space=pltpu.SMEM),
        pl.BlockSpec(memory_space=pltpu.VMEM),
        pl.BlockSpec(memory_space=pltpu.VMEM),
        pl.BlockSpec(memory_space=pltpu.VMEM),
        pl.BlockSpec(memory_space=pltpu.VMEM),
        pl.BlockSpec(memory_space=pltpu.SMEM),
    ],
    out_shape=[
        jax.ShapeDtypeStruct((NB, DD), jnp.float32),
        jax.ShapeDtypeStruct((NB, DD), jnp.float32),
        jax.ShapeDtypeStruct((NB, DD), jnp.float32),
    ],
)


def kernel(x, edge_index, candidates, action, machine_state,
           g0w1, g0b1, g0w2, g0b2, g1w1, g1b1, g1w2, g1b2,
           g2w1, g2b1, g2w2, g2b2, aw1, ab1, aw2, ab2, aw3, ab3,
           cw1, cb1, cw2, cb2):
    src = edge_index[0]
    dst = edge_index[1]
    offs = (jnp.arange(NB, dtype=jnp.int32) * NN)[:, None]
    srcg = (src[None, :] + offs).reshape(NB, TILES, NSEG, SEG, C)
    dstt = dst.reshape(TILES, NSEG, SEG, C)
    zrow = jnp.zeros((DRC, DD), jnp.float32)
    ms = machine_state.reshape(1, DD)

    h = x
    gmean = None
    for (w1, b1, w2, b2) in ((g0w1, g0b1, g0w2, g0b2),
                             (g1w1, g1b1, g1w2, g1b2),
                             (g2w1, g2b1, g2w2, g2b2)):
        agg = _segsum(h.reshape(NB * NN, DD), srcg, dstt, zrow)
        h, gmean = _gin(h, agg, w1, b1.reshape(1, DD), w2, b2.reshape(1, DD))

    s3 = _scores(h, gmean, ms, aw1, ab1.reshape(1, DD), aw2,
                 ab2.reshape(1, DD), aw3.reshape(1, DD), ab3)
    lps, ents, vs = _final(s3.reshape(NB, NN), candidates, action,
                           gmean.reshape(NB, DD), cw1, cb1.reshape(1, CH),
                           cw2.reshape(1, CH), cb2)
    return action, lps[:, 0], ents[:, 0], vs[:, 0]
